# baseline (speedup 1.0000x reference)
import functools

import jax
import jax.numpy as jnp
from jax import lax
from jax.experimental import pallas as pl
from jax.experimental.pallas import tpu as pltpu
from jax.experimental.pallas import tpu_sc as plsc

N = 10000
E = 320000
F_IN = 128
C = 64
CP = 128

NC = 2
NS = 16
NW = NC * NS

K = 128
EPAD = 327680
WPW = EPAD // (NW * K)
TOTWIN = EPAD // K
NPAD = 10240
RPT = NPAD // NS
CHUNK = 128
DEG_Q = 8

_MESH = plsc.VectorSubcoreMesh(core_axis_name="c", subcore_axis_name="s")


@functools.partial(
    pl.kernel,
    out_type=jax.ShapeDtypeStruct((NC, NPAD, CP), jnp.float32),
    mesh=_MESH,
    scratch_types=[
        pltpu.VMEM((WPW, K), jnp.int32),
        pltpu.VMEM((K, CP), jnp.float32),
        pltpu.VMEM((CHUNK, CP), jnp.float32),
        pltpu.VMEM_SHARED((NPAD, CP), jnp.float32),
        pltpu.SemaphoreType.DMA,
    ],
)
def _deg_partials(roww_hbm, ones_hbm, zeros_hbm, out_hbm,
                  idx_v, ones_v, buf_v, acc_sh, sem):
    c = lax.axis_index("c")
    s = lax.axis_index("s")
    w = c * NS + s
    pltpu.sync_copy(roww_hbm.at[pl.ds(w * WPW, WPW), :], idx_v)
    pltpu.sync_copy(ones_hbm, ones_v)
    pltpu.sync_copy(zeros_hbm, buf_v)
    for j in range(RPT // CHUNK):
        pltpu.sync_copy(buf_v, acc_sh.at[pl.ds(s * RPT + j * CHUNK, CHUNK), :])
    plsc.subcore_barrier()

    for q in range(DEG_Q):
        pltpu.async_copy(ones_v, acc_sh.at[idx_v.at[q]], sem, add=True)

    def body(t, carry):
        pltpu.make_async_copy(ones_v, acc_sh.at[idx_v.at[t]], sem).wait()
        nxt = jnp.minimum(t + DEG_Q, WPW - 1)

        @pl.when(t + DEG_Q < WPW)
        def _():
            pltpu.async_copy(ones_v, acc_sh.at[idx_v.at[nxt]], sem, add=True)

        return carry

    lax.fori_loop(0, WPW, body, 0)
    plsc.subcore_barrier()
    for j in range(RPT // CHUNK):
        r0 = s * RPT + j * CHUNK
        pltpu.sync_copy(acc_sh.at[pl.ds(r0, CHUNK), :], buf_v)
        pltpu.sync_copy(buf_v, out_hbm.at[c, pl.ds(r0, CHUNK), :])


@functools.partial(
    pl.kernel,
    out_type=jax.ShapeDtypeStruct((NC, NPAD, CP), jnp.float32),
    mesh=_MESH,
    scratch_types=[
        pltpu.VMEM((WPW // 2, K), jnp.int32),
        pltpu.VMEM((WPW // 2, K), jnp.int32),
        pltpu.VMEM((K, CP), jnp.float32),
        pltpu.VMEM((K, CP), jnp.float32),
        pltpu.VMEM_SHARED((NPAD, CP), jnp.float32),
        pltpu.SemaphoreType.DMA,
        pltpu.SemaphoreType.DMA,
        pltpu.SemaphoreType.DMA,
        pltpu.SemaphoreType.DMA,
    ],
)
def _aggregate(g_hbm, roww_hbm, colw_hbm, zeros_hbm, out_hbm,
               ridx_v, cidx_v, rows_a, rows_b, acc_sh,
               sem_ga, sem_gb, sem_sa, sem_sb):
    c = lax.axis_index("c")
    s = lax.axis_index("s")
    w = c * NS + s
    pltpu.sync_copy(zeros_hbm, rows_a)
    for j in range(RPT // CHUNK):
        pltpu.sync_copy(rows_a, acc_sh.at[pl.ds(s * RPT + j * CHUNK, CHUNK), :])
    plsc.subcore_barrier()

    def gather(i, rows, sem):
        return pltpu.async_copy(g_hbm.at[ridx_v.at[i]], rows, sem)

    def gather_wait(i, rows, sem):
        pltpu.make_async_copy(g_hbm.at[ridx_v.at[i]], rows, sem).wait()

    def scat(i, rows, sem):
        return pltpu.async_copy(rows, acc_sh.at[cidx_v.at[i]], sem, add=True)

    def scat_wait(i, rows, sem):
        pltpu.make_async_copy(rows, acc_sh.at[cidx_v.at[i]], sem).wait()

    HALF = WPW // 2
    for h in range(2):
        w0 = w * WPW + h * HALF
        pltpu.sync_copy(roww_hbm.at[pl.ds(w0, HALF), :], ridx_v)
        pltpu.sync_copy(colw_hbm.at[pl.ds(w0, HALF), :], cidx_v)
        gather(0, rows_a, sem_ga)

        def body(t, carry):
            i0 = 2 * t
            i1 = 2 * t + 1
            i2 = jnp.minimum(2 * t + 2, HALF - 1)
            gather_wait(i0, rows_a, sem_ga)
            scat(i0, rows_a, sem_sa)
            gather(i1, rows_b, sem_gb)
            scat_wait(i0, rows_a, sem_sa)
            gather_wait(i1, rows_b, sem_gb)
            scat(i1, rows_b, sem_sb)
            gather(i2, rows_a, sem_ga)
            scat_wait(i1, rows_b, sem_sb)
            return carry

        lax.fori_loop(0, HALF // 2, body, 0)
        gather_wait(HALF - 1, rows_a, sem_ga)
    plsc.subcore_barrier()
    for j in range(RPT // CHUNK):
        r0 = s * RPT + j * CHUNK
        pltpu.sync_copy(acc_sh.at[pl.ds(r0, CHUNK), :], rows_a)
        pltpu.sync_copy(rows_a, out_hbm.at[c, pl.ds(r0, CHUNK), :])


BLK = 1000


def _dinv_block(dp_ref):
    deg = dp_ref[0] + dp_ref[1]
    return jnp.where(deg > 0.0, lax.rsqrt(deg), 0.0)[:, :1]


def _mm_body(x_ref, w_ref, o_ref):
    o_ref[...] = jnp.dot(x_ref[...], w_ref[...],
                         preferred_element_type=jnp.float32)


def _scale0_body(dp_ref, h_ref, o_ref):
    o_ref[...] = h_ref[...] * _dinv_block(dp_ref)


def _combine_scale_mid_body(dp_ref, s_ref, o_ref):
    dinv = _dinv_block(dp_ref)
    o_ref[...] = (s_ref[0] + s_ref[1]) * (dinv * dinv)


def _combine_scale_out_body(dp_ref, s_ref, o_ref):
    dinv = _dinv_block(dp_ref)
    o_ref[...] = ((s_ref[0] + s_ref[1]) * dinv)[:, :C]


_dp_spec = pl.BlockSpec((NC, BLK, CP), lambda j: (0, j, 0))
_s_spec = pl.BlockSpec((NC, BLK, CP), lambda j: (0, j, 0))

_mm = pl.pallas_call(
    _mm_body,
    grid=(N // BLK,),
    in_specs=[
        pl.BlockSpec((BLK, F_IN), lambda j: (j, 0)),
        pl.BlockSpec((F_IN, CP), lambda j: (0, 0)),
    ],
    out_specs=pl.BlockSpec((BLK, CP), lambda j: (j, 0)),
    out_shape=jax.ShapeDtypeStruct((NPAD, CP), jnp.float32),
)

_scale0 = pl.pallas_call(
    _scale0_body,
    grid=(N // BLK,),
    in_specs=[
        _dp_spec,
        pl.BlockSpec((BLK, CP), lambda j: (j, 0)),
    ],
    out_specs=pl.BlockSpec((BLK, CP), lambda j: (j, 0)),
    out_shape=jax.ShapeDtypeStruct((NPAD, CP), jnp.float32),
)

_combine_scale_mid = pl.pallas_call(
    _combine_scale_mid_body,
    grid=(N // BLK,),
    in_specs=[_dp_spec, _s_spec],
    out_specs=pl.BlockSpec((BLK, CP), lambda j: (j, 0)),
    out_shape=jax.ShapeDtypeStruct((NPAD, CP), jnp.float32),
)

_combine_scale_out = pl.pallas_call(
    _combine_scale_out_body,
    grid=(N // BLK,),
    in_specs=[_dp_spec, _s_spec],
    out_specs=pl.BlockSpec((BLK, C), lambda j: (j, 0)),
    out_shape=jax.ShapeDtypeStruct((N, C), jnp.float32),
)


def kernel(x, edge_index, weight):
    row = edge_index[0]
    col = edge_index[1]
    pad = N + (jnp.arange(EPAD - E, dtype=jnp.int32) % (NPAD - N))
    roww = jnp.concatenate([row, pad]).reshape(TOTWIN, K)
    colw = jnp.concatenate([col, pad]).reshape(TOTWIN, K)
    wp = jnp.zeros((F_IN, CP), jnp.float32).at[:, :C].set(weight)
    onesC = jnp.ones((K, CP), jnp.float32)
    zerosC = jnp.zeros((CHUNK, CP), jnp.float32)

    dp = _deg_partials(roww, onesC, zerosC)
    h0 = _mm(x, wp)
    g0 = _scale0(dp, h0)
    s1 = _aggregate(g0, roww, colw, zerosC)
    g1 = _combine_scale_mid(dp, s1)
    s2 = _aggregate(g1, roww, colw, zerosC)
    return _combine_scale_out(dp, s2)


# speedup vs baseline: 1.0013x; 1.0013x over previous
import functools

import jax
import jax.numpy as jnp
from jax import lax
from jax.experimental import pallas as pl
from jax.experimental.pallas import tpu as pltpu
from jax.experimental.pallas import tpu_sc as plsc

N = 10000
E = 320000
F_IN = 128
C = 64
CP = 128

NC = 2
NS = 16
NW = NC * NS

K = 128
EPAD = 327680
WPW = EPAD // (NW * K)
TOTWIN = EPAD // K
NPAD = 10240
RPT = NPAD // NS
CHUNK = 128
DEG_Q = 8

_MESH = plsc.VectorSubcoreMesh(core_axis_name="c", subcore_axis_name="s")


@functools.partial(
    pl.kernel,
    out_type=jax.ShapeDtypeStruct((NC, NPAD, CP), jnp.float32),
    mesh=_MESH,
    scratch_types=[
        pltpu.VMEM((WPW, K), jnp.int32),
        pltpu.VMEM((K, CP), jnp.float32),
        pltpu.VMEM((CHUNK, CP), jnp.float32),
        pltpu.VMEM_SHARED((NPAD, CP), jnp.float32),
        pltpu.SemaphoreType.DMA,
    ],
)
def _deg_partials(roww_hbm, ones_hbm, zeros_hbm, out_hbm,
                  idx_v, ones_v, buf_v, acc_sh, sem):
    c = lax.axis_index("c")
    s = lax.axis_index("s")
    w = c * NS + s
    pltpu.sync_copy(roww_hbm.at[pl.ds(w * WPW, WPW), :], idx_v)
    pltpu.sync_copy(ones_hbm, ones_v)
    pltpu.sync_copy(zeros_hbm, buf_v)
    for j in range(RPT // CHUNK):
        pltpu.sync_copy(buf_v, acc_sh.at[pl.ds(s * RPT + j * CHUNK, CHUNK), :])
    plsc.subcore_barrier()

    for q in range(DEG_Q):
        pltpu.async_copy(ones_v, acc_sh.at[idx_v.at[q]], sem, add=True)

    def body(t, carry):
        pltpu.make_async_copy(ones_v, acc_sh.at[idx_v.at[t]], sem).wait()
        nxt = jnp.minimum(t + DEG_Q, WPW - 1)

        @pl.when(t + DEG_Q < WPW)
        def _():
            pltpu.async_copy(ones_v, acc_sh.at[idx_v.at[nxt]], sem, add=True)

        return carry

    lax.fori_loop(0, WPW, body, 0)
    plsc.subcore_barrier()
    for j in range(RPT // CHUNK):
        r0 = s * RPT + j * CHUNK
        pltpu.sync_copy(acc_sh.at[pl.ds(r0, CHUNK), :], buf_v)
        pltpu.sync_copy(buf_v, out_hbm.at[c, pl.ds(r0, CHUNK), :])


@functools.partial(
    pl.kernel,
    out_type=jax.ShapeDtypeStruct((NC, NPAD, CP), jnp.float32),
    mesh=_MESH,
    scratch_types=[
        pltpu.VMEM((WPW // 2, K), jnp.int32),
        pltpu.VMEM((WPW // 2, K), jnp.int32),
        pltpu.VMEM((K, CP), jnp.float32),
        pltpu.VMEM((K, CP), jnp.float32),
        pltpu.VMEM_SHARED((NPAD, CP), jnp.float32),
        pltpu.SemaphoreType.DMA,
        pltpu.SemaphoreType.DMA,
        pltpu.SemaphoreType.DMA,
        pltpu.SemaphoreType.DMA,
    ],
)
def _aggregate(g_hbm, roww_hbm, colw_hbm, zeros_hbm, out_hbm,
               ridx_v, cidx_v, rows_a, rows_b, acc_sh,
               sem_ga, sem_gb, sem_sa, sem_sb):
    c = lax.axis_index("c")
    s = lax.axis_index("s")
    w = c * NS + s
    pltpu.sync_copy(zeros_hbm, rows_a)
    for j in range(RPT // CHUNK):
        pltpu.sync_copy(rows_a, acc_sh.at[pl.ds(s * RPT + j * CHUNK, CHUNK), :])
    plsc.subcore_barrier()

    def gather(i, rows, sem):
        return pltpu.async_copy(g_hbm.at[ridx_v.at[i]], rows, sem)

    def gather_wait(i, rows, sem):
        pltpu.make_async_copy(g_hbm.at[ridx_v.at[i]], rows, sem).wait()

    def scat(i, rows, sem):
        return pltpu.async_copy(rows, acc_sh.at[cidx_v.at[i]], sem, add=True)

    def scat_wait(i, rows, sem):
        pltpu.make_async_copy(rows, acc_sh.at[cidx_v.at[i]], sem).wait()

    HALF = WPW // 2
    for h in range(2):
        w0 = w * WPW + h * HALF
        pltpu.sync_copy(roww_hbm.at[pl.ds(w0, HALF), :], ridx_v)
        pltpu.sync_copy(colw_hbm.at[pl.ds(w0, HALF), :], cidx_v)
        gather(0, rows_a, sem_ga)

        def body(t, carry):
            i0 = 2 * t
            i1 = 2 * t + 1
            i2 = jnp.minimum(2 * t + 2, HALF - 1)
            gather_wait(i0, rows_a, sem_ga)
            scat(i0, rows_a, sem_sa)
            gather(i1, rows_b, sem_gb)
            gather_wait(i1, rows_b, sem_gb)
            scat(i1, rows_b, sem_sb)
            scat_wait(i0, rows_a, sem_sa)
            gather(i2, rows_a, sem_ga)
            scat_wait(i1, rows_b, sem_sb)
            return carry

        lax.fori_loop(0, HALF // 2, body, 0)
        gather_wait(HALF - 1, rows_a, sem_ga)
    plsc.subcore_barrier()
    for j in range(RPT // CHUNK):
        r0 = s * RPT + j * CHUNK
        pltpu.sync_copy(acc_sh.at[pl.ds(r0, CHUNK), :], rows_a)
        pltpu.sync_copy(rows_a, out_hbm.at[c, pl.ds(r0, CHUNK), :])


BLK = 1000


def _dinv_block(dp_ref):
    deg = dp_ref[0] + dp_ref[1]
    return jnp.where(deg > 0.0, lax.rsqrt(deg), 0.0)[:, :1]


def _mm_scale_body(dp_ref, x_ref, w_ref, o_ref):
    h = jnp.dot(x_ref[...], w_ref[...], preferred_element_type=jnp.float32)
    o_ref[...] = h * _dinv_block(dp_ref)


def _combine_scale_mid_body(dp_ref, s_ref, o_ref):
    dinv = _dinv_block(dp_ref)
    o_ref[...] = (s_ref[0] + s_ref[1]) * (dinv * dinv)


def _combine_scale_out_body(dp_ref, s_ref, o_ref):
    dinv = _dinv_block(dp_ref)
    o_ref[...] = ((s_ref[0] + s_ref[1]) * dinv)[:, :C]


_dp_spec = pl.BlockSpec((NC, BLK, CP), lambda j: (0, j, 0))
_s_spec = pl.BlockSpec((NC, BLK, CP), lambda j: (0, j, 0))

_mm_scale = pl.pallas_call(
    _mm_scale_body,
    grid=(N // BLK,),
    in_specs=[
        _dp_spec,
        pl.BlockSpec((BLK, F_IN), lambda j: (j, 0)),
        pl.BlockSpec((F_IN, CP), lambda j: (0, 0)),
    ],
    out_specs=pl.BlockSpec((BLK, CP), lambda j: (j, 0)),
    out_shape=jax.ShapeDtypeStruct((NPAD, CP), jnp.float32),
)

_combine_scale_mid = pl.pallas_call(
    _combine_scale_mid_body,
    grid=(N // BLK,),
    in_specs=[_dp_spec, _s_spec],
    out_specs=pl.BlockSpec((BLK, CP), lambda j: (j, 0)),
    out_shape=jax.ShapeDtypeStruct((NPAD, CP), jnp.float32),
)

_combine_scale_out = pl.pallas_call(
    _combine_scale_out_body,
    grid=(N // BLK,),
    in_specs=[_dp_spec, _s_spec],
    out_specs=pl.BlockSpec((BLK, C), lambda j: (j, 0)),
    out_shape=jax.ShapeDtypeStruct((N, C), jnp.float32),
)


def kernel(x, edge_index, weight):
    row = edge_index[0]
    col = edge_index[1]
    pad = N + (jnp.arange(EPAD - E, dtype=jnp.int32) % (NPAD - N))
    roww = jnp.concatenate([row, pad]).reshape(TOTWIN, K)
    colw = jnp.concatenate([col, pad]).reshape(TOTWIN, K)
    wp = jnp.zeros((F_IN, CP), jnp.float32).at[:, :C].set(weight)
    onesC = jnp.ones((K, CP), jnp.float32)
    zerosC = jnp.zeros((CHUNK, CP), jnp.float32)

    dp = _deg_partials(roww, onesC, zerosC)
    g0 = _mm_scale(dp, x, wp)
    s1 = _aggregate(g0, roww, colw, zerosC)
    g1 = _combine_scale_mid(dp, s1)
    s2 = _aggregate(g1, roww, colw, zerosC)
    return _combine_scale_out(dp, s2)
